# baseline (speedup 1.0000x reference)
import functools

import numpy as np
import jax
import jax.numpy as jnp
from jax import lax
from jax.experimental import pallas as pl
from jax.experimental.pallas import tpu as pltpu
from jax.experimental.pallas import tpu_sc as plsc

ALPHA = 35.0
SIGMA = 2.5
B, D, H, W = 2, 128, 128, 128
N = B * D * H * W
NC, NS = 2, 16
NW = NC * NS
ROWS = B * D * H
RPW = ROWS // NW


def _gauss_taps():
    ks = int(2 * np.round(3 * SIGMA) + 1)
    half = ks // 2
    ax = np.arange(-half + 1, half + 1, dtype=np.float32)
    k = np.exp(-ax ** 2 / (2.0 * SIGMA ** 2))
    k = k / k.sum()
    return jnp.asarray(k, dtype=jnp.float32)


def _conv1(x, f):
    dn = jax.lax.conv_dimension_numbers(x.shape, f.shape,
                                        ('NDHWC', 'DHWIO', 'NDHWC'))
    return jax.lax.conv_general_dilated(x, f, (1, 1, 1), 'SAME',
                                        dimension_numbers=dn)


def _smooth3d(x, kern):
    L = kern.shape[0]
    x = _conv1(x, kern.reshape(L, 1, 1, 1, 1))
    x = _conv1(x, kern.reshape(1, L, 1, 1, 1))
    x = _conv1(x, kern.reshape(1, 1, L, 1, 1))
    return x


def _full_flow(coarse_flow):
    flow = jax.image.resize(coarse_flow, (B, D, H, W, 3), method='cubic')
    kern = _gauss_taps()
    comps = [_smooth3d(flow[..., i:i + 1], kern)[..., 0] for i in range(3)]
    return jnp.stack(comps, axis=-1) * ALPHA


def _floor_i32(x):
    i = x.astype(jnp.int32)
    fi = i.astype(jnp.float32)
    return i - jnp.where(fi > x, 1, 0)


def _round_half_even_i32(x):
    y = x + 0.5
    r = _floor_i32(y)
    tie = r.astype(jnp.float32) == y
    odd = (r & 1) == 1
    return r - jnp.where(tie & odd, 1, 0)


def _clampD(i):
    return jnp.maximum(jnp.minimum(i, D - 1), 0)


@functools.cache
def _make_sc_deform():
    mesh = plsc.VectorSubcoreMesh(core_axis_name="c", subcore_axis_name="s",
                                  num_cores=NC, num_subcores=NS)
    return functools.partial(
        pl.kernel,
        out_type=(jax.ShapeDtypeStruct((N,), jnp.float32),
                  jax.ShapeDtypeStruct((N,), jnp.float32)),
        mesh=mesh,
        scratch_types=[
            pltpu.VMEM((3, W), jnp.float32),
            pltpu.VMEM((9, W), jnp.int32),
            pltpu.VMEM((9, W), jnp.float32),
            pltpu.VMEM((3, W), jnp.float32),
            pltpu.VMEM((W,), jnp.float32),
            pltpu.SemaphoreType.DMA,
        ],
        compiler_params=pltpu.CompilerParams(needs_layout_passes=False),
    )(_sc_deform_body)


def _sc_deform_body(flow_hbm, img_hbm, lbl_hbm, oimg_hbm, olbl_hbm,
                    fl_v, idx_v, val_v, tw_v, outrow_v, sem):
    wid = lax.axis_index("s") * NC + lax.axis_index("c")

    def row_body(j, carry):
        r = wid * RPW + j
        h = lax.bitwise_and(r, H - 1)
        d = lax.bitwise_and(lax.shift_right_logical(r, 7), D - 1)
        n = lax.shift_right_logical(r, 14)
        rb = r * W
        nbase = n * (D * H * W)
        df = d.astype(jnp.float32)
        hf = h.astype(jnp.float32)

        for i in range(3):
            pltpu.sync_copy(flow_hbm.at[pl.ds(i * N + rb, W)], fl_v.at[i])

        for wc in range(8):
            sl = pl.ds(wc * 16, 16)
            wio = (lax.iota(jnp.int32, 16) + (wc * 16)).astype(jnp.float32)
            wd = fl_v[0, sl] + df
            wh = fl_v[1, sl] + hf
            ww = fl_v[2, sl] + wio
            d0 = _floor_i32(wd)
            h0 = _floor_i32(wh)
            w0 = _floor_i32(ww)
            tw_v[0, sl] = wd - d0.astype(jnp.float32)
            tw_v[1, sl] = wh - h0.astype(jnp.float32)
            tw_v[2, sl] = ww - w0.astype(jnp.float32)
            pd0 = lax.shift_left(_clampD(d0), 14)
            pd1 = lax.shift_left(_clampD(d0 + 1), 14)
            ph0 = lax.shift_left(_clampD(h0), 7)
            ph1 = lax.shift_left(_clampD(h0 + 1), 7)
            w0c = _clampD(w0)
            w1c = _clampD(w0 + 1)
            e00 = nbase + pd0 + ph0
            e01 = nbase + pd0 + ph1
            e10 = nbase + pd1 + ph0
            e11 = nbase + pd1 + ph1
            idx_v[0, sl] = e00 + w0c
            idx_v[1, sl] = e00 + w1c
            idx_v[2, sl] = e01 + w0c
            idx_v[3, sl] = e01 + w1c
            idx_v[4, sl] = e10 + w0c
            idx_v[5, sl] = e10 + w1c
            idx_v[6, sl] = e11 + w0c
            idx_v[7, sl] = e11 + w1c
            rd = _clampD(_round_half_even_i32(wd))
            rh = _clampD(_round_half_even_i32(wh))
            rw = _clampD(_round_half_even_i32(ww))
            idx_v[8, sl] = (nbase + lax.shift_left(rd, 14)
                            + lax.shift_left(rh, 7) + rw)

        copies = [pltpu.async_copy(img_hbm.at[idx_v.at[c]],
                                   val_v.at[c], sem) for c in range(8)]
        copies.append(pltpu.async_copy(lbl_hbm.at[idx_v.at[8]],
                                       val_v.at[8], sem))
        for cp in copies:
            cp.wait()

        for wc in range(8):
            sl = pl.ds(wc * 16, 16)
            td = tw_v[0, sl]
            th = tw_v[1, sl]
            tw = tw_v[2, sl]
            c000 = val_v[0, sl]
            c001 = val_v[1, sl]
            c010 = val_v[2, sl]
            c011 = val_v[3, sl]
            c100 = val_v[4, sl]
            c101 = val_v[5, sl]
            c110 = val_v[6, sl]
            c111 = val_v[7, sl]
            c00 = c000 + tw * (c001 - c000)
            c01 = c010 + tw * (c011 - c010)
            c10 = c100 + tw * (c101 - c100)
            c11 = c110 + tw * (c111 - c110)
            c0 = c00 + th * (c01 - c00)
            c1 = c10 + th * (c11 - c10)
            outrow_v[sl] = c0 + td * (c1 - c0)

        pltpu.sync_copy(outrow_v, oimg_hbm.at[pl.ds(rb, W)])
        pltpu.sync_copy(val_v.at[8], olbl_hbm.at[pl.ds(rb, W)])
        return carry

    lax.fori_loop(0, RPW, row_body, 0)


def kernel(image_volume, label_volume, coarse_flow):
    img = image_volume.reshape(N)
    lbl = label_volume.reshape(N)
    flow = _full_flow(coarse_flow.astype(jnp.float32))
    flow3 = jnp.moveaxis(flow, -1, 0).reshape(3 * N)
    oimg, olbl = _make_sc_deform()(flow3, img, lbl)
    return (oimg.reshape(B, D, H, W, 1), olbl.reshape(B, D, H, W, 1))


# speedup vs baseline: 6.0591x; 6.0591x over previous
import functools

import numpy as np
import jax
import jax.numpy as jnp
from jax import lax
from jax.experimental import pallas as pl
from jax.experimental.pallas import tpu as pltpu
from jax.experimental.pallas import tpu_sc as plsc

ALPHA = 35.0
SIGMA = 2.5
B, D, H, W = 2, 128, 128, 128
N = B * D * H * W
NC, NS = 2, 16
NW = NC * NS
ROWS = B * D * H
RPW = ROWS // NW

_DN = lax.ConvDimensionNumbers(lhs_spec=(2, 4, 0, 1, 3),
                               rhs_spec=(4, 3, 0, 1, 2),
                               out_spec=(2, 4, 0, 1, 3))


def _gauss_taps():
    ks = int(2 * np.round(3 * SIGMA) + 1)
    half = ks // 2
    ax = np.arange(-half + 1, half + 1, dtype=np.float32)
    k = np.exp(-ax ** 2 / (2.0 * SIGMA ** 2))
    k = k / k.sum()
    return jnp.asarray(k, dtype=jnp.float32)


def _smooth3d(x, kern):
    L = kern.shape[0]
    kd = kern.reshape(L, 1, 1, 1, 1)
    kh = kern.reshape(1, L, 1, 1, 1)
    kw = kern.reshape(1, 1, L, 1, 1)
    xt = x.transpose(1, 2, 0, 3, 4).reshape(D, H, 16, 16, 1)
    y = lax.conv_general_dilated(xt, kd, (1, 1, 1),
                                 [(7, 8), (0, 0), (0, 0)],
                                 dimension_numbers=_DN)
    y = lax.conv_general_dilated(y, kh, (1, 1, 1),
                                 [(0, 0), (7, 8), (0, 0)],
                                 dimension_numbers=_DN)
    z = y.reshape(D, H, 2, 8, 16, 1).transpose(2, 0, 1, 3, 4, 5)
    z = z.reshape(2, D, H, W, 1)
    zp = jnp.pad(z, ((0, 0), (0, 0), (0, 0), (7, 8), (0, 0)))
    chunks = jnp.stack([zp[:, :, :, c * 16:c * 16 + 31, :]
                        for c in range(8)], axis=3)
    ct = chunks.transpose(1, 2, 0, 3, 4, 5).reshape(D, H, 16, 31, 1)
    w = lax.conv_general_dilated(ct, kw, (1, 1, 1),
                                 [(0, 0), (0, 0), (0, 0)],
                                 dimension_numbers=_DN)
    out = w.reshape(D, H, 2, 8, 16, 1).transpose(2, 0, 1, 3, 4, 5)
    return out.reshape(2, D, H, W, 1)


def _full_flow(coarse_flow):
    flow = jax.image.resize(coarse_flow, (B, D, H, W, 3), method='cubic')
    kern = _gauss_taps()
    comps = [_smooth3d(flow[..., i:i + 1], kern)[..., 0] for i in range(3)]
    return jnp.stack(comps, axis=-1) * ALPHA


def _floor_i32(x):
    i = x.astype(jnp.int32)
    fi = i.astype(jnp.float32)
    return i - jnp.where(fi > x, 1, 0)


def _round_half_even_i32(x):
    y = x + 0.5
    r = _floor_i32(y)
    tie = r.astype(jnp.float32) == y
    odd = (r & 1) == 1
    return r - jnp.where(tie & odd, 1, 0)


def _clampD(i):
    return jnp.maximum(jnp.minimum(i, D - 1), 0)


@functools.cache
def _make_sc_deform():
    mesh = plsc.VectorSubcoreMesh(core_axis_name="c", subcore_axis_name="s",
                                  num_cores=NC, num_subcores=NS)
    return functools.partial(
        pl.kernel,
        out_type=(jax.ShapeDtypeStruct((N,), jnp.float32),
                  jax.ShapeDtypeStruct((N,), jnp.float32)),
        mesh=mesh,
        scratch_types=[
            pltpu.VMEM((3 * W,), jnp.float32),
            pltpu.VMEM((9, W), jnp.int32),
            pltpu.VMEM((9, W), jnp.float32),
            pltpu.VMEM((3, W), jnp.float32),
            pltpu.VMEM((W,), jnp.float32),
            pltpu.SemaphoreType.DMA,
        ],
        compiler_params=pltpu.CompilerParams(needs_layout_passes=False),
    )(_sc_deform_body)


def _sc_deform_body(flow_hbm, img_hbm, lbl_hbm, oimg_hbm, olbl_hbm,
                    fl_v, idx_v, val_v, tw_v, outrow_v, sem):
    wid = lax.axis_index("s") * NC + lax.axis_index("c")
    i3 = lax.iota(jnp.int32, 16) * 3

    def row_body(j, carry):
        r = wid * RPW + j
        h = lax.bitwise_and(r, H - 1)
        d = lax.bitwise_and(lax.shift_right_logical(r, 7), D - 1)
        n = lax.shift_right_logical(r, 14)
        rb = r * W
        nbase = n * (D * H * W)
        df = d.astype(jnp.float32)
        hf = h.astype(jnp.float32)
        pltpu.sync_copy(flow_hbm.at[pl.ds(rb * 3, 3 * W)], fl_v)

        for wc in range(8):
            sl = pl.ds(wc * 16, 16)
            wio = (lax.iota(jnp.int32, 16) + (wc * 16)).astype(jnp.float32)
            base = i3 + (wc * 48)
            wd = plsc.load_gather(fl_v, [base]) + df
            wh = plsc.load_gather(fl_v, [base + 1]) + hf
            ww = plsc.load_gather(fl_v, [base + 2]) + wio
            d0 = _floor_i32(wd)
            h0 = _floor_i32(wh)
            w0 = _floor_i32(ww)
            tw_v[0, sl] = wd - d0.astype(jnp.float32)
            tw_v[1, sl] = wh - h0.astype(jnp.float32)
            tw_v[2, sl] = ww - w0.astype(jnp.float32)
            pd0 = lax.shift_left(_clampD(d0), 14)
            pd1 = lax.shift_left(_clampD(d0 + 1), 14)
            ph0 = lax.shift_left(_clampD(h0), 7)
            ph1 = lax.shift_left(_clampD(h0 + 1), 7)
            w0c = _clampD(w0)
            w1c = _clampD(w0 + 1)
            e00 = nbase + pd0 + ph0
            e01 = nbase + pd0 + ph1
            e10 = nbase + pd1 + ph0
            e11 = nbase + pd1 + ph1
            idx_v[0, sl] = e00 + w0c
            idx_v[1, sl] = e00 + w1c
            idx_v[2, sl] = e01 + w0c
            idx_v[3, sl] = e01 + w1c
            idx_v[4, sl] = e10 + w0c
            idx_v[5, sl] = e10 + w1c
            idx_v[6, sl] = e11 + w0c
            idx_v[7, sl] = e11 + w1c
            rd = _clampD(_round_half_even_i32(wd))
            rh = _clampD(_round_half_even_i32(wh))
            rw = _clampD(_round_half_even_i32(ww))
            idx_v[8, sl] = (nbase + lax.shift_left(rd, 14)
                            + lax.shift_left(rh, 7) + rw)

        copies = [pltpu.async_copy(img_hbm.at[idx_v.at[c]],
                                   val_v.at[c], sem) for c in range(8)]
        copies.append(pltpu.async_copy(lbl_hbm.at[idx_v.at[8]],
                                       val_v.at[8], sem))
        for cp in copies:
            cp.wait()

        for wc in range(8):
            sl = pl.ds(wc * 16, 16)
            td = tw_v[0, sl]
            th = tw_v[1, sl]
            tw = tw_v[2, sl]
            c000 = val_v[0, sl]
            c001 = val_v[1, sl]
            c010 = val_v[2, sl]
            c011 = val_v[3, sl]
            c100 = val_v[4, sl]
            c101 = val_v[5, sl]
            c110 = val_v[6, sl]
            c111 = val_v[7, sl]
            c00 = c000 + tw * (c001 - c000)
            c01 = c010 + tw * (c011 - c010)
            c10 = c100 + tw * (c101 - c100)
            c11 = c110 + tw * (c111 - c110)
            c0 = c00 + th * (c01 - c00)
            c1 = c10 + th * (c11 - c10)
            outrow_v[sl] = c0 + td * (c1 - c0)

        pltpu.sync_copy(outrow_v, oimg_hbm.at[pl.ds(rb, W)])
        pltpu.sync_copy(val_v.at[8], olbl_hbm.at[pl.ds(rb, W)])
        return carry

    lax.fori_loop(0, RPW, row_body, 0)


def kernel(image_volume, label_volume, coarse_flow):
    img = image_volume.reshape(N)
    lbl = label_volume.reshape(N)
    flow = _full_flow(coarse_flow.astype(jnp.float32))
    f3 = flow.reshape(3 * N)
    oimg, olbl = _make_sc_deform()(f3, img, lbl)
    return (oimg.reshape(B, D, H, W, 1), olbl.reshape(B, D, H, W, 1))
